# baseline (speedup 1.0000x reference)
import functools

import jax
import jax.numpy as jnp
from jax import lax
from jax.experimental import pallas as pl
from jax.experimental.pallas import tpu as pltpu
from jax.experimental.pallas import tpu_sc as plsc

NUM_CORES = 2
NUM_SUBCORES = 16
NW = NUM_CORES * NUM_SUBCORES
CHUNK = 128


def _build(n_chunks, D):
    mesh = plsc.VectorSubcoreMesh(core_axis_name="c", subcore_axis_name="s")

    @functools.partial(
        pl.kernel,
        mesh=mesh,
        out_type=jax.ShapeDtypeStruct((NW, n_chunks, CHUNK, D), jnp.float32),
        scratch_types=[
            pltpu.VMEM((n_chunks, CHUNK), jnp.int32),
            pltpu.VMEM((CHUNK, D), jnp.float32),
            pltpu.SemaphoreType.DMA,
        ],
        compiler_params=pltpu.CompilerParams(use_tc_tiling_on_sc=False),
    )
    def k(table_hbm, y_hbm, out_hbm, idx_v, rows_v, gsem):
        wid = lax.axis_index("s") * NUM_CORES + lax.axis_index("c")
        pltpu.sync_copy(y_hbm.at[wid], idx_v)

        def body(j, carry):
            pltpu.async_copy(table_hbm.at[idx_v.at[j]], rows_v, gsem).wait()
            pltpu.sync_copy(rows_v, out_hbm.at[wid, j])
            return carry

        lax.fori_loop(0, n_chunks, body, 0)

    return k


@jax.jit
def kernel(y, table):
    B, L = y.shape
    D = table.shape[1]
    n = B * L
    per_w = n // NW
    n_chunks = per_w // CHUNK
    y3 = y.reshape(NW, n_chunks, CHUNK)
    out = _build(n_chunks, D)(table, y3)
    return out.reshape(B, L, D)


# speedup vs baseline: 1.1123x; 1.1123x over previous
import functools

import jax
import jax.numpy as jnp
from jax import lax
from jax.experimental import pallas as pl
from jax.experimental.pallas import tpu as pltpu
from jax.experimental.pallas import tpu_sc as plsc

NUM_CORES = 2
NUM_SUBCORES = 16
NW = NUM_CORES * NUM_SUBCORES
CHUNK = 128
K = 5


def _build(n_super, D):
    n_chunks = n_super * K
    rows_per_super = K * CHUNK
    mesh = plsc.VectorSubcoreMesh(core_axis_name="c", subcore_axis_name="s")

    @functools.partial(
        pl.kernel,
        mesh=mesh,
        out_type=jax.ShapeDtypeStruct((NW, n_super, rows_per_super, D), jnp.float32),
        scratch_types=[
            pltpu.VMEM((n_chunks, CHUNK), jnp.int32),
            pltpu.VMEM((2, rows_per_super, D), jnp.float32),
            pltpu.SemaphoreType.DMA,
            pltpu.SemaphoreType.DMA,
            pltpu.SemaphoreType.DMA,
            pltpu.SemaphoreType.DMA,
        ],
        compiler_params=pltpu.CompilerParams(use_tc_tiling_on_sc=False),
    )
    def k(table_hbm, y_hbm, out_hbm, idx_v, rows_v, g0, g1, o0, o1):
        wid = lax.axis_index("s") * NUM_CORES + lax.axis_index("c")
        gsem = (g0, g1)
        osem = (o0, o1)
        pltpu.sync_copy(y_hbm.at[wid], idx_v)

        def fire(s, h):
            for kk in range(K):
                pltpu.async_copy(
                    table_hbm.at[idx_v.at[s * K + kk]],
                    rows_v.at[h, pl.ds(kk * CHUNK, CHUNK)],
                    gsem[h],
                )

        def drain_gathers(s, h):
            pltpu.make_async_copy(out_hbm.at[wid, s], rows_v.at[h], gsem[h]).wait()

        fire(0, 0)

        def body(s2, carry):
            for h in range(2):
                s = s2 * 2 + h

                @pl.when(s + 1 < n_super)
                def _():
                    @pl.when(s >= 1)
                    def _():
                        pltpu.make_async_copy(
                            rows_v.at[1 - h], out_hbm.at[wid, s - 1], osem[1 - h]
                        ).wait()

                    fire(s + 1, 1 - h)

                drain_gathers(s, h)
                pltpu.async_copy(rows_v.at[h], out_hbm.at[wid, s], osem[h])
            return carry

        lax.fori_loop(0, n_super // 2, body, 0)
        pltpu.make_async_copy(
            rows_v.at[0], out_hbm.at[wid, n_super - 2], osem[0]
        ).wait()
        pltpu.make_async_copy(
            rows_v.at[1], out_hbm.at[wid, n_super - 1], osem[1]
        ).wait()

    return k


@jax.jit
def kernel(y, table):
    B, L = y.shape
    D = table.shape[1]
    n = B * L
    per_w = n // NW
    n_super = per_w // (K * CHUNK)
    y3 = y.reshape(NW, n_super * K, CHUNK)
    out = _build(n_super, D)(table, y3)
    return out.reshape(B, L, D)


# speedup vs baseline: 1.1138x; 1.0014x over previous
import functools

import jax
import jax.numpy as jnp
from jax import lax
from jax.experimental import pallas as pl
from jax.experimental.pallas import tpu as pltpu
from jax.experimental.pallas import tpu_sc as plsc

NUM_CORES = 2
NUM_SUBCORES = 16
NW = NUM_CORES * NUM_SUBCORES
CHUNK = 128
NBUF = 4


def _build(B, L, D):
    rows_per_w = B // NW
    rest = L - CHUNK
    mesh = plsc.VectorSubcoreMesh(core_axis_name="c", subcore_axis_name="s")

    @functools.partial(
        pl.kernel,
        mesh=mesh,
        out_type=jax.ShapeDtypeStruct((B, L, D), jnp.float32),
        scratch_types=[
            pltpu.VMEM((rows_per_w, CHUNK), jnp.int32),
            pltpu.VMEM((rows_per_w, rest), jnp.int32),
            pltpu.VMEM((NBUF, L, D), jnp.float32),
            [pltpu.SemaphoreType.DMA] * NBUF,
            [pltpu.SemaphoreType.DMA] * NBUF,
        ],
        compiler_params=pltpu.CompilerParams(use_tc_tiling_on_sc=False),
    )
    def k(y_hbm, table_hbm, out_hbm, idx_a, idx_b, rows_v, gsems, osems):
        wid = lax.axis_index("s") * NUM_CORES + lax.axis_index("c")
        base = wid * rows_per_w
        pltpu.sync_copy(y_hbm.at[pl.ds(base, rows_per_w), pl.ds(0, CHUNK)], idx_a)
        pltpu.sync_copy(y_hbm.at[pl.ds(base, rows_per_w), pl.ds(CHUNK, rest)], idx_b)

        def fire(i, d):
            pltpu.async_copy(
                table_hbm.at[idx_a.at[i]], rows_v.at[d, pl.ds(0, CHUNK)], gsems[d]
            )
            pltpu.async_copy(
                table_hbm.at[idx_b.at[i]], rows_v.at[d, pl.ds(CHUNK, rest)], gsems[d]
            )

        def drain_gathers(i, d):
            pltpu.make_async_copy(out_hbm.at[base + i], rows_v.at[d], gsems[d]).wait()

        for d in range(NBUF):
            fire(d, d)

        def body(g, carry):
            for d in range(NBUF):
                i = g * NBUF + d
                drain_gathers(i, d)
                pltpu.async_copy(rows_v.at[d], out_hbm.at[base + i], osems[d])

                @pl.when(i + NBUF < rows_per_w)
                def _():
                    pltpu.make_async_copy(
                        rows_v.at[d], out_hbm.at[base + i], osems[d]
                    ).wait()
                    fire(i + NBUF, d)

            return carry

        lax.fori_loop(0, rows_per_w // NBUF, body, 0)
        for d in range(NBUF):
            i = rows_per_w - NBUF + d
            pltpu.make_async_copy(rows_v.at[d], out_hbm.at[base + i], osems[d]).wait()

    return k


@jax.jit
def kernel(y, table):
    B, L = y.shape
    D = table.shape[1]
    return _build(B, L, D)(y, table)


# speedup vs baseline: 1.4813x; 1.3299x over previous
import functools

import jax
import jax.numpy as jnp
from jax import lax
from jax.experimental import pallas as pl
from jax.experimental.pallas import tpu as pltpu
from jax.experimental.pallas import tpu_sc as plsc

NUM_CORES = 2
NUM_SUBCORES = 16
NW = NUM_CORES * NUM_SUBCORES
CHUNK = 128
NBUF = 4


def _build(B, L, D):
    rows_per_w = B // NW
    rest = L - CHUNK
    mesh = plsc.VectorSubcoreMesh(core_axis_name="c", subcore_axis_name="s")

    @functools.partial(
        pl.kernel,
        mesh=mesh,
        out_type=jax.ShapeDtypeStruct((B, L, 2 * D), jnp.float32),
        scratch_types=[
            pltpu.VMEM((rows_per_w, CHUNK), jnp.int32),
            pltpu.VMEM((rows_per_w, rest), jnp.int32),
            pltpu.VMEM((NBUF, L, D), jnp.float32),
            [pltpu.SemaphoreType.DMA] * NBUF,
            [pltpu.SemaphoreType.DMA] * NBUF,
        ],
        compiler_params=pltpu.CompilerParams(use_tc_tiling_on_sc=False),
    )
    def k(y_hbm, table_hbm, out_hbm, idx_a, idx_b, rows_v, gsems, osems):
        wid = lax.axis_index("s") * NUM_CORES + lax.axis_index("c")
        base = wid * rows_per_w
        pltpu.sync_copy(y_hbm.at[pl.ds(base, rows_per_w), pl.ds(0, CHUNK)], idx_a)
        pltpu.sync_copy(y_hbm.at[pl.ds(base, rows_per_w), pl.ds(CHUNK, rest)], idx_b)

        def fire(i, d):
            pltpu.async_copy(
                table_hbm.at[idx_a.at[i]], rows_v.at[d, pl.ds(0, CHUNK)], gsems[d]
            )
            pltpu.async_copy(
                table_hbm.at[idx_b.at[i]], rows_v.at[d, pl.ds(CHUNK, rest)], gsems[d]
            )

        def drain_gathers(i, d):
            pltpu.make_async_copy(out_hbm.at[base + i, pl.ds(0, L), pl.ds(0, D)], rows_v.at[d], gsems[d]).wait()

        for d in range(NBUF):
            fire(d, d)

        def body(g, carry):
            for d in range(NBUF):
                i = g * NBUF + d
                drain_gathers(i, d)
                pltpu.async_copy(rows_v.at[d], out_hbm.at[base + i, pl.ds(0, L), pl.ds(0, D)], osems[d])

                @pl.when(i + NBUF < rows_per_w)
                def _():
                    pltpu.make_async_copy(
                        rows_v.at[d], out_hbm.at[base + i, pl.ds(0, L), pl.ds(0, D)],
                        osems[d],
                    ).wait()
                    fire(i + NBUF, d)

            return carry

        lax.fori_loop(0, rows_per_w // NBUF, body, 0)
        for d in range(NBUF):
            i = rows_per_w - NBUF + d
            pltpu.make_async_copy(
                rows_v.at[d], out_hbm.at[base + i, pl.ds(0, L), pl.ds(0, D)],
                osems[d],
            ).wait()

    return k


@jax.jit
def kernel(y, table):
    B, L = y.shape
    D = table.shape[1]
    return _build(B, L, D)(y, table)[:, :, :D]
